# Initial kernel scaffold; baseline (speedup 1.0000x reference)
import jax
import jax.numpy as jnp
from jax.experimental import pallas as pl


def kernel(x, edge_index, W_enc, b_enc, W_bases, W_comb, b_conv, W_ih, W_hh, b_ih, b_hh, W1, b1, W2, b2):
    raise NotImplementedError("write your pallas kernel here")


# speedup vs baseline: 3.3789x; 3.3789x over previous
import functools

import jax
import jax.numpy as jnp
from jax import lax
from jax.experimental import pallas as pl
from jax.experimental.pallas import tpu as pltpu

_N_NODES = 10000
_IN_ENC = 128
_OUT_ENC = 64
_OUT_GNN = 64
_NPS = 100
_SAMPLE_NUM = 100
_HIDDEN = 512
_FC1 = 256
_NUM_CLASSES = 10
_NUM_HEADS = 8
_NUM_BASES = 4
_T = 4
_THRESH = 0.3
_DECAY = 0.2
_ENC_THRESH = 0.3
_BW = _OUT_GNN // _NUM_HEADS
_NB32 = _NUM_BASES * _BW

_RB = 1000
_GRID_N = _N_NODES // _RB

_MMD = jnp.bfloat16


def _enc_body(xs_ref, mem_ref, dinv_ref, wenc_ref, wbc_ref, benc_ref,
              mem_out, bs_out, comb_out):
    mem_p = mem_ref[...]
    spk_p = (mem_p > _ENC_THRESH).astype(jnp.float32)
    xt = xs_ref[0]
    z = jnp.dot(xt.astype(_MMD), wenc_ref[...],
                preferred_element_type=jnp.float32)
    mem = mem_p * _DECAY * (1.0 - spk_p) + z + benc_ref[...]
    mem_out[...] = mem
    spk = (mem > _ENC_THRESH).astype(jnp.float32)
    bc = jnp.dot(spk.astype(_MMD), wbc_ref[...],
                 preferred_element_type=jnp.float32)
    bs_out[...] = bc[:, :_NB32] * dinv_ref[...]
    comb_out[...] = bc[:, _NB32:]


def _enc_step(x_t, enc_mem, dinv, wenc_t, wbc_t, benc):
    return pl.pallas_call(
        _enc_body,
        grid=(_GRID_N,),
        in_specs=[
            pl.BlockSpec((1, _RB, _IN_ENC), lambda i: (0, i, 0)),
            pl.BlockSpec((_RB, _OUT_ENC), lambda i: (i, 0)),
            pl.BlockSpec((_RB, 1), lambda i: (i, 0)),
            pl.BlockSpec((_IN_ENC, _OUT_ENC), lambda i: (0, 0)),
            pl.BlockSpec((_OUT_ENC, 2 * _NB32), lambda i: (0, 0)),
            pl.BlockSpec((1, _OUT_ENC), lambda i: (0, 0)),
        ],
        out_specs=[
            pl.BlockSpec((_RB, _OUT_ENC), lambda i: (i, 0)),
            pl.BlockSpec((_RB, _NB32), lambda i: (i, 0)),
            pl.BlockSpec((_RB, _NB32), lambda i: (i, 0)),
        ],
        out_shape=[
            jax.ShapeDtypeStruct((_N_NODES, _OUT_ENC), jnp.float32),
            jax.ShapeDtypeStruct((_N_NODES, _NB32), jnp.float32),
            jax.ShapeDtypeStruct((_N_NODES, _NB32), jnp.float32),
        ],
    )(x_t, enc_mem, dinv, wenc_t, wbc_t, benc)


def _combine_body(acc_ref, bs_ref, dinv_ref, comb_ref, c1mem_ref, bconv_ref,
                  c1mem_out, c1spk_out):
    agg = (acc_ref[...] + bs_ref[...]) * dinv_ref[...]
    conv = jnp.zeros((_RB, _OUT_GNN), jnp.float32)
    for b in range(_NUM_BASES):
        cb = comb_ref[:, b * _BW:(b + 1) * _BW]
        ab = agg[:, b * _BW:(b + 1) * _BW]
        ch = jnp.repeat(cb, _BW, axis=1)
        af = jnp.concatenate([ab] * _NUM_HEADS, axis=1)
        conv = conv + ch * af
    conv = conv + bconv_ref[...]
    mem_p = c1mem_ref[...]
    spk_p = (mem_p > _THRESH).astype(jnp.float32)
    mem = mem_p * _DECAY * (1.0 - spk_p) + conv
    c1mem_out[...] = mem
    c1spk_out[...] = (mem > _THRESH).astype(jnp.float32)


def _combine_step(acc, bs, dinv, comb, c1mem, bconv):
    return pl.pallas_call(
        _combine_body,
        grid=(_GRID_N,),
        in_specs=[
            pl.BlockSpec((_RB, _NB32), lambda i: (i, 0)),
            pl.BlockSpec((_RB, _NB32), lambda i: (i, 0)),
            pl.BlockSpec((_RB, 1), lambda i: (i, 0)),
            pl.BlockSpec((_RB, _NB32), lambda i: (i, 0)),
            pl.BlockSpec((_RB, _OUT_GNN), lambda i: (i, 0)),
            pl.BlockSpec((1, _OUT_GNN), lambda i: (0, 0)),
        ],
        out_specs=[
            pl.BlockSpec((_RB, _OUT_GNN), lambda i: (i, 0)),
            pl.BlockSpec((_RB, _OUT_GNN), lambda i: (i, 0)),
        ],
        out_shape=[
            jax.ShapeDtypeStruct((_N_NODES, _OUT_GNN), jnp.float32),
            jax.ShapeDtypeStruct((_N_NODES, _OUT_GNN), jnp.float32),
        ],
    )(acc, bs, dinv, comb, c1mem, bconv)


_HB = 128
_GRID_H = _HIDDEN // _HB


def _lstm_body(xf_ref, h_ref, c_ref, wih_ref, whh_ref, bih_ref, bhh_ref,
               h_out, c_out):
    xf = xf_ref[...].astype(_MMD)
    hp = h_ref[...].astype(_MMD)
    dn = (((1,), (1,)), ((), ()))
    gates = []
    for g in range(4):
        zx = lax.dot_general(xf, wih_ref[g], dn,
                             preferred_element_type=jnp.float32)
        zh = lax.dot_general(hp, whh_ref[g], dn,
                             preferred_element_type=jnp.float32)
        gate = zx + bih_ref[g][None, :] + zh + bhh_ref[g][None, :]
        gates.append((gate > 0.0).astype(jnp.float32))
    i_s, f_s, g_s, o_s = gates
    c_new = f_s * c_ref[...] + i_s * g_s
    c_out[...] = c_new
    h_out[...] = o_s * (c_new > 0.0).astype(jnp.float32)


def _lstm_step(xf, h, c, wih_r, whh_r, bih_r, bhh_r):
    k = _HIDDEN * 4 * _NPS // 32
    return pl.pallas_call(
        _lstm_body,
        grid=(_GRID_H,),
        in_specs=[
            pl.BlockSpec((_SAMPLE_NUM, _NPS * _OUT_GNN), lambda i: (0, 0)),
            pl.BlockSpec((_SAMPLE_NUM, _HIDDEN), lambda i: (0, 0)),
            pl.BlockSpec((_SAMPLE_NUM, _HB), lambda i: (0, i)),
            pl.BlockSpec((4, _HB, _NPS * _OUT_GNN), lambda i: (0, i, 0)),
            pl.BlockSpec((4, _HB, _HIDDEN), lambda i: (0, i, 0)),
            pl.BlockSpec((4, _HB), lambda i: (0, i)),
            pl.BlockSpec((4, _HB), lambda i: (0, i)),
        ],
        out_specs=[
            pl.BlockSpec((_SAMPLE_NUM, _HB), lambda i: (0, i)),
            pl.BlockSpec((_SAMPLE_NUM, _HB), lambda i: (0, i)),
        ],
        out_shape=[
            jax.ShapeDtypeStruct((_SAMPLE_NUM, _HIDDEN), jnp.float32),
            jax.ShapeDtypeStruct((_SAMPLE_NUM, _HIDDEN), jnp.float32),
        ],
    )(xf, h, c, wih_r, whh_r, bih_r, bhh_r)


def _head_body(lh_ref, w1_ref, b1_ref, h1mem_ref, w2_ref, b2_ref,
               h2mem_ref, h2sum_ref, h1mem_out, h2mem_out, h2sum_out):
    dn = (((1,), (1,)), ((), ()))
    z1 = lax.dot_general(lh_ref[...].astype(_MMD), w1_ref[...], dn,
                         preferred_element_type=jnp.float32)
    m1p = h1mem_ref[...]
    s1p = (m1p > _THRESH).astype(jnp.float32)
    m1 = m1p * _DECAY * (1.0 - s1p) + z1 + b1_ref[...]
    h1mem_out[...] = m1
    s1 = (m1 > _THRESH).astype(jnp.float32)
    z2 = lax.dot_general(s1.astype(_MMD), w2_ref[...], dn,
                         preferred_element_type=jnp.float32)
    m2p = h2mem_ref[...]
    s2p = (m2p > _THRESH).astype(jnp.float32)
    m2 = m2p * _DECAY * (1.0 - s2p) + z2 + b2_ref[...]
    h2mem_out[...] = m2
    s2 = (m2 > _THRESH).astype(jnp.float32)
    h2sum_out[...] = h2sum_ref[...] + s2


def _head_step(lh, w1_bf, b1, h1mem, w2_bf, b2, h2mem, h2sum):
    return pl.pallas_call(
        _head_body,
        out_shape=[
            jax.ShapeDtypeStruct((_SAMPLE_NUM, _FC1), jnp.float32),
            jax.ShapeDtypeStruct((_SAMPLE_NUM, _NUM_CLASSES), jnp.float32),
            jax.ShapeDtypeStruct((_SAMPLE_NUM, _NUM_CLASSES), jnp.float32),
        ],
    )(lh, w1_bf, b1, h1mem, w2_bf, b2, h2mem, h2sum)


def kernel(x, edge_index, W_enc, b_enc, W_bases, W_comb, b_conv,
           W_ih, W_hh, b_ih, b_hh, W1, b1, W2, b2):
    src = edge_index[0]
    dst = edge_index[1]

    deg = jnp.ones((_N_NODES,), jnp.float32).at[dst].add(1.0)
    dinv = lax.rsqrt(jnp.maximum(deg, 1.0))
    dinv2 = dinv[:, None]

    x_t = jnp.transpose(x, (2, 0, 1))
    wenc_t = W_enc.T.astype(_MMD)
    perm = jnp.array([h * _NUM_BASES + b
                      for b in range(_NUM_BASES)
                      for h in range(_NUM_HEADS)], jnp.int32)
    wbc_t = jnp.concatenate([W_bases.T, W_comb[perm].T], axis=1).astype(_MMD)
    benc = b_enc[None, :]
    bconv = b_conv[None, :]
    wih_r = W_ih.reshape(4, _HIDDEN, _NPS * _OUT_GNN).astype(_MMD)
    whh_r = W_hh.reshape(4, _HIDDEN, _HIDDEN).astype(_MMD)
    bih_r = b_ih.reshape(4, _HIDDEN)
    bhh_r = b_hh.reshape(4, _HIDDEN)
    w1_bf = W1.astype(_MMD)
    w2_bf = W2.astype(_MMD)
    b1r = b1[None, :]
    b2r = b2[None, :]

    enc_mem = jnp.zeros((_N_NODES, _OUT_ENC), jnp.float32)
    c1mem = jnp.zeros((_N_NODES, _OUT_GNN), jnp.float32)
    lh = jnp.zeros((_SAMPLE_NUM, _HIDDEN), jnp.float32)
    lc = jnp.zeros((_SAMPLE_NUM, _HIDDEN), jnp.float32)
    h1mem = jnp.zeros((_SAMPLE_NUM, _FC1), jnp.float32)
    h2mem = jnp.zeros((_SAMPLE_NUM, _NUM_CLASSES), jnp.float32)
    h2sum = jnp.zeros((_SAMPLE_NUM, _NUM_CLASSES), jnp.float32)

    for step in range(_T):
        enc_mem, bs, comb = _enc_step(
            lax.dynamic_slice_in_dim(x_t, step, 1, axis=0),
            enc_mem, dinv2, wenc_t, wbc_t, benc)
        acc = jnp.zeros((_N_NODES, _NB32), jnp.float32).at[dst].add(bs[src])
        c1mem, c1spk = _combine_step(acc, bs, dinv2, comb, c1mem, bconv)
        xf = c1spk.reshape(_SAMPLE_NUM, _NPS * _OUT_GNN)
        lh, lc = _lstm_step(xf, lh, lc, wih_r, whh_r, bih_r, bhh_r)
        h1mem, h2mem, h2sum = _head_step(
            lh, w1_bf, b1r, h1mem, w2_bf, b2r, h2mem, h2sum)

    return h2sum / float(_T)
